# baseline (speedup 1.0000x reference)
import functools

import jax
import jax.numpy as jnp
from jax import lax
from jax.experimental import pallas as pl
from jax.experimental.pallas import tpu as pltpu
from jax.experimental.pallas import tpu_sc as plsc

N = 50000
D = 256
H = 256
B = 512
S = 100

NW = 32
CHUNK = 1568
STEPS = CHUNK // 16
LAST = N - 31 * CHUNK
LAST_STEPS = LAST // 16
ACC_W = 528
BLK = 8192
GRID = pl.cdiv(N, BLK)
SPAD = 128


def _mlp_body(x_ref, w1_ref, b1_ref, w2_ref, o_ref):
    xb = x_ref[...].astype(jnp.bfloat16)
    h = jnp.dot(xb, w1_ref[...], preferred_element_type=jnp.float32)
    h = jnp.maximum(h + b1_ref[...], 0.0).astype(jnp.bfloat16)
    e8 = jnp.dot(h, w2_ref[...], preferred_element_type=jnp.float32)
    o_ref[...] = jnp.sum(e8, axis=1)


def _mlp(x, W1, b1r, w2p):
    return pl.pallas_call(
        _mlp_body,
        grid=(GRID,),
        in_specs=[
            pl.BlockSpec((BLK, D), lambda i: (i, 0)),
            pl.BlockSpec((D, H), lambda i: (0, 0)),
            pl.BlockSpec((1, H), lambda i: (0, 0)),
            pl.BlockSpec((H, 8), lambda i: (0, 0)),
        ],
        out_specs=pl.BlockSpec((BLK,), lambda i: (i,)),
        out_shape=jax.ShapeDtypeStruct((N,), jnp.float32),
        compiler_params=pltpu.CompilerParams(
            dimension_semantics=("arbitrary",)),
    )(x, W1, b1r, w2p)


def _sc_body(e_hbm, z_hbm, bt_hbm, sc_hbm, sh_hbm, out_hbm,
             e_v, z_v, bt_v, sc_v, sh_v, acc, part, big, shared):
    cid = lax.axis_index("c")
    sid = lax.axis_index("s")
    wid = cid * 16 + sid
    base = wid * CHUNK

    @pl.when(wid < NW - 1)
    def _():
        pltpu.sync_copy(e_hbm.at[pl.ds(base, CHUNK)], e_v)
        pltpu.sync_copy(z_hbm.at[pl.ds(base, CHUNK)], z_v)
        pltpu.sync_copy(bt_hbm.at[pl.ds(base, CHUNK)], bt_v)

    @pl.when(wid == NW - 1)
    def _():
        lo = (NW - 1) * CHUNK
        pltpu.sync_copy(e_hbm.at[pl.ds(lo, LAST)], e_v.at[pl.ds(0, LAST)])
        pltpu.sync_copy(z_hbm.at[pl.ds(lo, LAST)], z_v.at[pl.ds(0, LAST)])
        pltpu.sync_copy(bt_hbm.at[pl.ds(lo, LAST)], bt_v.at[pl.ds(0, LAST)])

    pltpu.sync_copy(sc_hbm, sc_v)
    pltpu.sync_copy(sh_hbm, sh_v)

    zeros16 = jnp.zeros((16,), jnp.float32)
    lanes = lax.iota(jnp.int32, 16)
    row_off = lanes * ACC_W

    def zero_body(j, c):
        acc[pl.ds(j * 16, 16)] = zeros16
        return c
    lax.fori_loop(0, 16 * ACC_W // 16, zero_body, 0)

    nsteps = jnp.where(wid == NW - 1, LAST_STEPS, STEPS)

    def step(j, c):
        off = j * 16
        ev = e_v[pl.ds(off, 16)]
        zv = z_v[pl.ds(off, 16)]
        bv = bt_v[pl.ds(off, 16)]
        s = plsc.load_gather(sc_v, [zv])
        sh = plsc.load_gather(sh_v, [zv])
        plsc.addupdate_scatter(acc, [row_off + bv], ev * s + sh)
        return c
    lax.fori_loop(0, nsteps, step, 0)

    def fold(j, c):
        col = j * 16
        v = acc[pl.ds(col, 16)]
        for r in range(1, 16):
            v = v + acc[pl.ds(r * ACC_W + col, 16)]
        part[pl.ds(col, 16)] = v
        return c
    lax.fori_loop(0, B // 16, fold, 0)

    pltpu.sync_copy(part, shared.at[sid])
    plsc.subcore_barrier()

    @pl.when(sid == 0)
    def _():
        for r in range(16):
            pltpu.sync_copy(shared.at[r], big.at[pl.ds(r * B, B)])

        def fold2(j, c):
            col = j * 16
            v = big[pl.ds(col, 16)]
            for r in range(1, 16):
                v = v + big[pl.ds(r * B + col, 16)]
            part[pl.ds(col, 16)] = v
            return c
        lax.fori_loop(0, B // 16, fold2, 0)
        pltpu.sync_copy(part, out_hbm.at[cid])


@functools.cache
def _sc_segsum():
  return functools.partial(
    pl.kernel,
    out_type=jax.ShapeDtypeStruct((2, B), jnp.float32),
    mesh=plsc.VectorSubcoreMesh(core_axis_name="c", subcore_axis_name="s",
                                num_cores=2, num_subcores=16),
    scratch_types=[
        pltpu.VMEM((CHUNK,), jnp.float32),
        pltpu.VMEM((CHUNK,), jnp.int32),
        pltpu.VMEM((CHUNK,), jnp.int32),
        pltpu.VMEM((SPAD,), jnp.float32),
        pltpu.VMEM((SPAD,), jnp.float32),
        pltpu.VMEM((16 * ACC_W,), jnp.float32),
        pltpu.VMEM((B,), jnp.float32),
        pltpu.VMEM((16 * B,), jnp.float32),
        pltpu.VMEM_SHARED((16, B), jnp.float32),
    ],
    compiler_params=pltpu.CompilerParams(needs_layout_passes=False),
  )(_sc_body)


def kernel(x, W1, b1, W2, b2, scale, shift, Z, batch):
    w2p = jnp.concatenate([W2, jnp.zeros((H, 7), jnp.float32)],
                          axis=1).astype(jnp.bfloat16)
    e = _mlp(x, W1.astype(jnp.bfloat16), b1.reshape(1, H), w2p)
    scp = jnp.pad(scale, (0, SPAD - S))
    shp = jnp.pad(shift + b2[0] * scale, (0, SPAD - S))
    parts = _sc_segsum()(e, Z.astype(jnp.int32), batch.astype(jnp.int32),
                         scp, shp)
    return parts[0] + parts[1]


# speedup vs baseline: 2.4021x; 2.4021x over previous
import functools

import jax
import jax.numpy as jnp
from jax import lax
from jax.experimental import pallas as pl
from jax.experimental.pallas import tpu as pltpu
from jax.experimental.pallas import tpu_sc as plsc

N = 50000
D = 256
H = 256
B = 512
S = 100

NW = 32
CHUNK = 1568
STEPS = CHUNK // 16
LAST = N - 31 * CHUNK
LAST_STEPS = LAST // 16
ACC_W = 528
BLK = 8192
GRID = pl.cdiv(N, BLK)
SPAD = 128


def _mlp_body(x_ref, w1_ref, b1_ref, w2_ref, o_ref):
    xb = x_ref[...].astype(jnp.bfloat16)
    hT = lax.dot_general(w1_ref[...], xb, (((0,), (1,)), ((), ())),
                         preferred_element_type=jnp.float32)
    hT = jnp.maximum(hT + b1_ref[...], 0.0)
    o_ref[...] = jnp.sum(hT * w2_ref[...], axis=0)


def _mlp(x, W1, b1r, w2p):
    return pl.pallas_call(
        _mlp_body,
        grid=(GRID,),
        in_specs=[
            pl.BlockSpec((BLK, D), lambda i: (i, 0)),
            pl.BlockSpec((D, H), lambda i: (0, 0)),
            pl.BlockSpec((H, 1), lambda i: (0, 0)),
            pl.BlockSpec((H, 1), lambda i: (0, 0)),
        ],
        out_specs=pl.BlockSpec((BLK,), lambda i: (i,)),
        out_shape=jax.ShapeDtypeStruct((N,), jnp.float32),
        compiler_params=pltpu.CompilerParams(
            dimension_semantics=("arbitrary",)),
    )(x, W1, b1r, w2p)


def _sc_body(e_hbm, z_hbm, bt_hbm, sc_hbm, sh_hbm, out_hbm,
             e_v, z_v, bt_v, sc_v, sh_v, acc, part, big, shared):
    cid = lax.axis_index("c")
    sid = lax.axis_index("s")
    wid = cid * 16 + sid
    base = wid * CHUNK

    @pl.when(wid < NW - 1)
    def _():
        pltpu.sync_copy(e_hbm.at[pl.ds(base, CHUNK)], e_v)
        pltpu.sync_copy(z_hbm.at[pl.ds(base, CHUNK)], z_v)
        pltpu.sync_copy(bt_hbm.at[pl.ds(base, CHUNK)], bt_v)

    @pl.when(wid == NW - 1)
    def _():
        lo = (NW - 1) * CHUNK
        pltpu.sync_copy(e_hbm.at[pl.ds(lo, LAST)], e_v.at[pl.ds(0, LAST)])
        pltpu.sync_copy(z_hbm.at[pl.ds(lo, LAST)], z_v.at[pl.ds(0, LAST)])
        pltpu.sync_copy(bt_hbm.at[pl.ds(lo, LAST)], bt_v.at[pl.ds(0, LAST)])

    pltpu.sync_copy(sc_hbm, sc_v)
    pltpu.sync_copy(sh_hbm, sh_v)

    zeros16 = jnp.zeros((16,), jnp.float32)
    lanes = lax.iota(jnp.int32, 16)
    row_off = lanes * ACC_W

    def zero_body(j, c):
        acc[pl.ds(j * 16, 16)] = zeros16
        return c
    lax.fori_loop(0, 16 * ACC_W // 16, zero_body, 0)

    nsteps = jnp.where(wid == NW - 1, LAST_STEPS, STEPS)

    def step(j, c):
        off = j * 16
        ev = e_v[pl.ds(off, 16)]
        zv = z_v[pl.ds(off, 16)]
        bv = bt_v[pl.ds(off, 16)]
        s = plsc.load_gather(sc_v, [zv])
        sh = plsc.load_gather(sh_v, [zv])
        plsc.addupdate_scatter(acc, [row_off + bv], ev * s + sh)
        return c
    lax.fori_loop(0, nsteps, step, 0)

    def fold(j, c):
        col = j * 16
        v = acc[pl.ds(col, 16)]
        for r in range(1, 16):
            v = v + acc[pl.ds(r * ACC_W + col, 16)]
        part[pl.ds(col, 16)] = v
        return c
    lax.fori_loop(0, B // 16, fold, 0)

    pltpu.sync_copy(part, shared.at[sid])
    plsc.subcore_barrier()

    @pl.when(sid == 0)
    def _():
        for r in range(16):
            pltpu.sync_copy(shared.at[r], big.at[pl.ds(r * B, B)])

        def fold2(j, c):
            col = j * 16
            v = big[pl.ds(col, 16)]
            for r in range(1, 16):
                v = v + big[pl.ds(r * B + col, 16)]
            part[pl.ds(col, 16)] = v
            return c
        lax.fori_loop(0, B // 16, fold2, 0)
        pltpu.sync_copy(part, out_hbm.at[cid])


@functools.cache
def _sc_segsum():
  return functools.partial(
    pl.kernel,
    out_type=jax.ShapeDtypeStruct((2, B), jnp.float32),
    mesh=plsc.VectorSubcoreMesh(core_axis_name="c", subcore_axis_name="s",
                                num_cores=2, num_subcores=16),
    scratch_types=[
        pltpu.VMEM((CHUNK,), jnp.float32),
        pltpu.VMEM((CHUNK,), jnp.int32),
        pltpu.VMEM((CHUNK,), jnp.int32),
        pltpu.VMEM((SPAD,), jnp.float32),
        pltpu.VMEM((SPAD,), jnp.float32),
        pltpu.VMEM((16 * ACC_W,), jnp.float32),
        pltpu.VMEM((B,), jnp.float32),
        pltpu.VMEM((16 * B,), jnp.float32),
        pltpu.VMEM_SHARED((16, B), jnp.float32),
    ],
    compiler_params=pltpu.CompilerParams(needs_layout_passes=False),
  )(_sc_body)


def kernel(x, W1, b1, W2, b2, scale, shift, Z, batch):
    e = _mlp(x, W1.astype(jnp.bfloat16), b1.reshape(H, 1), W2)
    scp = jnp.pad(scale, (0, SPAD - S))
    shp = jnp.pad(shift + b2[0] * scale, (0, SPAD - S))
    parts = _sc_segsum()(e, Z.astype(jnp.int32), batch.astype(jnp.int32),
                         scp, shp)
    return parts[0] + parts[1]


# speedup vs baseline: 2.5864x; 1.0767x over previous
import functools

import jax
import jax.numpy as jnp
from jax import lax
from jax.experimental import pallas as pl
from jax.experimental.pallas import tpu as pltpu
from jax.experimental.pallas import tpu_sc as plsc

N = 50000
D = 256
H = 256
B = 512
S = 100

NW = 32
CHUNK = 1568
STEPS = CHUNK // 16
LAST = N - 31 * CHUNK
LAST_STEPS = LAST // 16
ACC_W = 528
BLK = 8192
GRID = pl.cdiv(N, BLK)
SPAD = 128


def _mlp_body(x_ref, w1_ref, aux_ref, b2_ref, o_ref):
    xb = x_ref[...].astype(jnp.bfloat16)
    hT = lax.dot_general(w1_ref[...], xb, (((0,), (1,)), ((), ())),
                         preferred_element_type=jnp.float32)
    hT = jnp.maximum(hT + aux_ref[:, 0:1], 0.0)
    o_ref[...] = jnp.sum(hT * aux_ref[:, 1:2], axis=0) + b2_ref[0, 0]


def _mlp(x, W1, aux, b2r):
    return pl.pallas_call(
        _mlp_body,
        grid=(GRID,),
        in_specs=[
            pl.BlockSpec((BLK, D), lambda i: (i, 0)),
            pl.BlockSpec((D, H), lambda i: (0, 0)),
            pl.BlockSpec((H, 2), lambda i: (0, 0)),
            pl.BlockSpec(memory_space=pltpu.SMEM),
        ],
        out_specs=pl.BlockSpec((BLK,), lambda i: (i,)),
        out_shape=jax.ShapeDtypeStruct((N,), jnp.float32),
        compiler_params=pltpu.CompilerParams(
            dimension_semantics=("arbitrary",)),
    )(x, W1, aux, b2r)


def _sc_body(e_hbm, z_hbm, bt_hbm, sc_hbm, sh_hbm, out_hbm,
             e_v, z_v, bt_v, sc_v, sh_v, acc, part, big, shared):
    cid = lax.axis_index("c")
    sid = lax.axis_index("s")
    wid = cid * 16 + sid
    base = wid * CHUNK

    @pl.when(wid < NW - 1)
    def _():
        pltpu.sync_copy(e_hbm.at[pl.ds(base, CHUNK)], e_v)
        pltpu.sync_copy(z_hbm.at[pl.ds(base, CHUNK)], z_v)
        pltpu.sync_copy(bt_hbm.at[pl.ds(base, CHUNK)], bt_v)

    @pl.when(wid == NW - 1)
    def _():
        lo = (NW - 1) * CHUNK
        pltpu.sync_copy(e_hbm.at[pl.ds(lo, LAST)], e_v.at[pl.ds(0, LAST)])
        pltpu.sync_copy(z_hbm.at[pl.ds(lo, LAST)], z_v.at[pl.ds(0, LAST)])
        pltpu.sync_copy(bt_hbm.at[pl.ds(lo, LAST)], bt_v.at[pl.ds(0, LAST)])

    pltpu.sync_copy(sc_hbm, sc_v.at[pl.ds(0, S)])
    pltpu.sync_copy(sh_hbm, sh_v.at[pl.ds(0, S)])

    zeros16 = jnp.zeros((16,), jnp.float32)
    lanes = lax.iota(jnp.int32, 16)
    row_off = lanes * ACC_W

    def zero_body(j, c):
        acc[pl.ds(j * 16, 16)] = zeros16
        return c
    lax.fori_loop(0, 16 * ACC_W // 16, zero_body, 0)

    nsteps = jnp.where(wid == NW - 1, LAST_STEPS, STEPS)

    def step(j, c):
        off = j * 16
        ev = e_v[pl.ds(off, 16)]
        zv = z_v[pl.ds(off, 16)]
        bv = bt_v[pl.ds(off, 16)]
        s = plsc.load_gather(sc_v, [zv])
        sh = plsc.load_gather(sh_v, [zv])
        plsc.addupdate_scatter(acc, [row_off + bv], ev * s + sh)
        return c
    lax.fori_loop(0, nsteps, step, 0)

    def fold(j, c):
        col = j * 16
        v = acc[pl.ds(col, 16)]
        for r in range(1, 16):
            v = v + acc[pl.ds(r * ACC_W + col, 16)]
        part[pl.ds(col, 16)] = v
        return c
    lax.fori_loop(0, B // 16, fold, 0)

    pltpu.sync_copy(part, shared.at[sid])
    plsc.subcore_barrier()

    @pl.when(sid == 0)
    def _():
        for r in range(16):
            pltpu.sync_copy(shared.at[r], big.at[pl.ds(r * B, B)])

        def fold2(j, c):
            col = j * 16
            v = big[pl.ds(col, 16)]
            for r in range(1, 16):
                v = v + big[pl.ds(r * B + col, 16)]
            part[pl.ds(col, 16)] = v
            return c
        lax.fori_loop(0, B // 16, fold2, 0)
        pltpu.sync_copy(part, out_hbm.at[cid])


@functools.cache
def _sc_segsum():
  return functools.partial(
    pl.kernel,
    out_type=jax.ShapeDtypeStruct((2, B), jnp.float32),
    mesh=plsc.VectorSubcoreMesh(core_axis_name="c", subcore_axis_name="s",
                                num_cores=2, num_subcores=16),
    scratch_types=[
        pltpu.VMEM((CHUNK,), jnp.float32),
        pltpu.VMEM((CHUNK,), jnp.int32),
        pltpu.VMEM((CHUNK,), jnp.int32),
        pltpu.VMEM((112,), jnp.float32),
        pltpu.VMEM((112,), jnp.float32),
        pltpu.VMEM((16 * ACC_W,), jnp.float32),
        pltpu.VMEM((B,), jnp.float32),
        pltpu.VMEM((16 * B,), jnp.float32),
        pltpu.VMEM_SHARED((16, B), jnp.float32),
    ],
    compiler_params=pltpu.CompilerParams(needs_layout_passes=False),
  )(_sc_body)


def kernel(x, W1, b1, W2, b2, scale, shift, Z, batch):
    aux = jnp.concatenate([b1.reshape(H, 1), W2], axis=1)
    e = _mlp(x, W1.astype(jnp.bfloat16), aux, b2.reshape(1, 1))
    parts = _sc_segsum()(e, Z.astype(jnp.int32), batch.astype(jnp.int32),
                         scale, shift)
    return parts[0] + parts[1]
